# Initial kernel scaffold; baseline (speedup 1.0000x reference)
import jax
import jax.numpy as jnp
from jax.experimental import pallas as pl


def kernel(x_disease, x_gene, src_dg, dst_dg, src_gd, dst_gd, W_l_dg1, W_r_dg1, W_l_gd1, W_r_gd1, W_l_dg2, W_r_dg2, W_l_gd2, W_r_gd2, b_dg1, b_gd1, b_dg2, b_gd2):
    raise NotImplementedError("write your pallas kernel here")


# speedup vs baseline: 2.5398x; 2.5398x over previous
import functools

import jax
import jax.numpy as jnp
from jax import lax
from jax.experimental import pallas as pl
from jax.experimental.pallas import tpu as pltpu
from jax.experimental.pallas import tpu_sc as plsc

N_D = 10000
N_G = 50000
E = 320000
D = 128

NC = 2
NS = 16

G_CHUNK = 12800
G_NCHUNK = 4
G_PAD = G_CHUNK * G_NCHUNK
G_GARB = 256
G_SH_ROWS = G_CHUNK + G_GARB

D_PAD = 10240

W_CNT = 2000

_mesh = plsc.VectorSubcoreMesh(core_axis_name="c", subcore_axis_name="s")

def _zero_vmem_2d(buf, rows):
    z = jnp.zeros((16,), jnp.float32)

    @pl.loop(0, rows)
    def _(r):
        for l in range(8):
            buf[r, pl.ds(l * 16, 16)] = z


def _fill_vmem_1d(buf, n, value):
    v = jnp.full((16,), value, jnp.float32)

    @pl.loop(0, n, step=16)
    def _(i):
        buf[pl.ds(i, 16)] = v


@functools.partial(
    pl.kernel,
    out_type=(
        jax.ShapeDtypeStruct((G_PAD,), jnp.float32),
        jax.ShapeDtypeStruct((D_PAD,), jnp.float32),
    ),
    mesh=_mesh,
    scratch_types=[
        pltpu.VMEM((3200,), jnp.float32),
        pltpu.VMEM((W_CNT,), jnp.float32),
        pltpu.VMEM((W_CNT,), jnp.int32),
        pltpu.VMEM_SHARED((G_PAD,), jnp.float32),
        pltpu.VMEM_SHARED((D_PAD,), jnp.float32),
    ],
)
def _sc_counts(dst_dg, dst_gd, cnt_g_out, cnt_d_out, zbuf, ones_v, idx_v,
               sh_g, sh_d):
    core = lax.axis_index("c")
    sid = lax.axis_index("s")
    _fill_vmem_1d(zbuf, 3200, 0.0)
    _fill_vmem_1d(ones_v, W_CNT, 1.0)

    @pl.when(core == 0)
    def _():
        pltpu.sync_copy(zbuf.at[pl.ds(0, G_PAD // NS)],
                        sh_g.at[pl.ds(sid * (G_PAD // NS), G_PAD // NS)])

    @pl.when(core == 1)
    def _():
        pltpu.sync_copy(zbuf.at[pl.ds(0, D_PAD // NS)],
                        sh_d.at[pl.ds(sid * (D_PAD // NS), D_PAD // NS)])

    plsc.subcore_barrier()

    per_tile = E // NS

    @pl.loop(0, per_tile // W_CNT)
    def _(w):
        base = sid * per_tile + w * W_CNT

        @pl.when(core == 0)
        def _():
            pltpu.sync_copy(dst_dg.at[pl.ds(base, W_CNT)], idx_v)
            pltpu.sync_copy(ones_v, sh_g.at[idx_v], add=True)

        @pl.when(core == 1)
        def _():
            pltpu.sync_copy(dst_gd.at[pl.ds(base, W_CNT)], idx_v)
            pltpu.sync_copy(ones_v, sh_d.at[idx_v], add=True)

    plsc.subcore_barrier()

    @pl.when(core == 0)
    def _():
        pltpu.sync_copy(sh_g.at[pl.ds(sid * (G_PAD // NS), G_PAD // NS)],
                        cnt_g_out.at[pl.ds(sid * (G_PAD // NS), G_PAD // NS)])

    @pl.when(core == 1)
    def _():
        pltpu.sync_copy(sh_d.at[pl.ds(sid * (D_PAD // NS), D_PAD // NS)],
                        cnt_d_out.at[pl.ds(sid * (D_PAD // NS), D_PAD // NS)])


W_GENE = 800
B_GENE = 160


@functools.partial(
    pl.kernel,
    out_type=jax.ShapeDtypeStruct((G_PAD, D), jnp.float32),
    mesh=_mesh,
    scratch_types=[
        pltpu.VMEM((32, D), jnp.float32),
        pltpu.VMEM((W_GENE,), jnp.int32),
        pltpu.VMEM((W_GENE,), jnp.int32),
        pltpu.VMEM((B_GENE,), jnp.int32),
        pltpu.VMEM((B_GENE, D), jnp.float32),
        pltpu.VMEM_SHARED((G_SH_ROWS, D), jnp.float32),
        pltpu.SemaphoreType.DMA,
    ],
)
def _sc_agg_gene(x_src, src_idx, dst_idx, acc_out, zbuf, src_v, dst_v, red_v,
                 rows_v, sh_acc, sem):
    core = lax.axis_index("c")
    sid = lax.axis_index("s")
    _zero_vmem_2d(zbuf, 32)

    per_tile = E // NS
    zrows = G_SH_ROWS // NS
    orows = G_CHUNK // NS
    lane = lax.iota(jnp.int32, 16)

    for ci in range(G_NCHUNK // NC):
        chunk = core + NC * ci
        lo = chunk * G_CHUNK

        @pl.loop(0, 800, step=32)
        def _(r):
            pltpu.sync_copy(zbuf.at[pl.ds(0, 32)],
                            sh_acc.at[pl.ds(sid * zrows + r, 32)])

        pltpu.sync_copy(zbuf.at[pl.ds(0, 16)],
                        sh_acc.at[pl.ds(sid * zrows + 800, 16)])

        plsc.subcore_barrier()

        @pl.loop(0, per_tile // W_GENE)
        def _(w):
            base = sid * per_tile + w * W_GENE
            pltpu.sync_copy(src_idx.at[pl.ds(base, W_GENE)], src_v)
            pltpu.sync_copy(dst_idx.at[pl.ds(base, W_GENE)], dst_v)

            @pl.loop(0, W_GENE // B_GENE)
            def _(s):
                pltpu.async_copy(x_src.at[src_v.at[pl.ds(s * B_GENE, B_GENE)]],
                                 rows_v, sem).wait()

                @pl.loop(0, B_GENE // 16)
                def _(j):
                    d16 = dst_v[pl.ds(s * B_GENE + j * 16, 16)]
                    keep = (d16 >= lo) & (d16 < lo + G_CHUNK)
                    gbase = ((sid + s * 10 + j) * 16) % G_GARB
                    garb = G_CHUNK + gbase + lane
                    red_v[pl.ds(j * 16, 16)] = jnp.where(keep, d16 - lo, garb)

                pltpu.sync_copy(rows_v, sh_acc.at[red_v], add=True)

        plsc.subcore_barrier()
        pltpu.sync_copy(sh_acc.at[pl.ds(sid * orows, orows)],
                        acc_out.at[pl.ds(lo + sid * orows, orows)])
        plsc.subcore_barrier()


W_DIS = 400
B_DIS = 80


@functools.partial(
    pl.kernel,
    out_type=jax.ShapeDtypeStruct((NC, D_PAD, D), jnp.float32),
    mesh=_mesh,
    scratch_types=[
        pltpu.VMEM((32, D), jnp.float32),
        pltpu.VMEM((W_DIS,), jnp.int32),
        pltpu.VMEM((W_DIS,), jnp.int32),
        pltpu.VMEM((B_DIS,), jnp.int32),
        pltpu.VMEM((B_DIS, D), jnp.float32),
        pltpu.VMEM_SHARED((D_PAD, D), jnp.float32),
        pltpu.SemaphoreType.DMA,
    ],
)
def _sc_agg_disease(x_src, src_idx, dst_idx, acc_out, zbuf, src_v, dst_v,
                    red_v, rows_v, sh_acc, sem):
    core = lax.axis_index("c")
    sid = lax.axis_index("s")
    _zero_vmem_2d(zbuf, 32)

    per_tile = E // (NC * NS)
    zrows = D_PAD // NS

    @pl.loop(0, zrows, step=32)
    def _(r):
        pltpu.sync_copy(zbuf.at[pl.ds(0, 32)],
                        sh_acc.at[pl.ds(sid * zrows + r, 32)])

    plsc.subcore_barrier()

    @pl.loop(0, per_tile // W_DIS)
    def _(w):
        base = core * (E // NC) + sid * per_tile + w * W_DIS
        pltpu.sync_copy(src_idx.at[pl.ds(base, W_DIS)], src_v)
        pltpu.sync_copy(dst_idx.at[pl.ds(base, W_DIS)], dst_v)

        @pl.loop(0, W_DIS // B_DIS)
        def _(s):
            pltpu.async_copy(x_src.at[src_v.at[pl.ds(s * B_DIS, B_DIS)]],
                             rows_v, sem).wait()

            @pl.loop(0, B_DIS // 16)
            def _(j):
                red_v[pl.ds(j * 16, 16)] = dst_v[pl.ds(s * B_DIS + j * 16, 16)]

            pltpu.sync_copy(rows_v, sh_acc.at[red_v], add=True)

    plsc.subcore_barrier()
    pltpu.sync_copy(sh_acc.at[pl.ds(sid * zrows, zrows)],
                    acc_out.at[core].at[pl.ds(sid * zrows, zrows)])


def _transform_body(two_acc, a0_ref, a1_ref, cnt_ref, x_ref, wl_ref, wr_ref,
                    b_ref, o_ref):
    if two_acc:
        agg = a0_ref[0] + a1_ref[0]
    else:
        agg = a0_ref[...]
        del a1_ref
    inv = 1.0 / jnp.maximum(cnt_ref[...], 1.0)
    agg = agg * inv
    dn = (((1,), (1,)), ((), ()))
    out = lax.dot_general(agg, wl_ref[...], dn,
                          precision=lax.Precision.HIGHEST,
                          preferred_element_type=jnp.float32)
    out += lax.dot_general(x_ref[...], wr_ref[...], dn,
                           precision=lax.Precision.HIGHEST,
                           preferred_element_type=jnp.float32)
    o_ref[...] = out + b_ref[...]


def _transform(n_rows, acc, partials, cnt, x_dst, w_l, w_r, b):
    bk = 2000
    grid = (n_rows // bk,)
    if partials:
        in_specs = [
            pl.BlockSpec((1, bk, D), lambda i: (0, i, 0)),
            pl.BlockSpec((1, bk, D), lambda i: (1, i, 0)),
        ]
        args = (acc, acc)
        body = functools.partial(_transform_body, True)
    else:
        in_specs = [
            pl.BlockSpec((bk, D), lambda i: (i, 0)),
        ]
        args = (acc,)

        def body(a0, cnt_r, x_r, wl, wr, b_r, o_r):
            _transform_body(False, a0, None, cnt_r, x_r, wl, wr, b_r, o_r)
    in_specs += [
        pl.BlockSpec((bk, 1), lambda i: (i, 0)),
        pl.BlockSpec((bk, D), lambda i: (i, 0)),
        pl.BlockSpec((D, D), lambda i: (0, 0)),
        pl.BlockSpec((D, D), lambda i: (0, 0)),
        pl.BlockSpec((1, D), lambda i: (0, 0)),
    ]
    return pl.pallas_call(
        body,
        grid=grid,
        in_specs=in_specs,
        out_specs=pl.BlockSpec((bk, D), lambda i: (i, 0)),
        out_shape=jax.ShapeDtypeStruct((n_rows, D), jnp.float32),
    )(*args, cnt, x_dst, w_l, w_r, b)


def kernel(x_disease, x_gene, src_dg, dst_dg, src_gd, dst_gd,
           W_l_dg1, W_r_dg1, W_l_gd1, W_r_gd1,
           W_l_dg2, W_r_dg2, W_l_gd2, W_r_gd2,
           b_dg1, b_gd1, b_dg2, b_gd2):
    cnt_g, cnt_d = _sc_counts(dst_dg, dst_gd)
    cnt_g = cnt_g.reshape(G_PAD, 1)
    cnt_d = cnt_d.reshape(D_PAD, 1)

    acc_g1 = _sc_agg_gene(x_disease, src_dg, dst_dg)
    acc_d1 = _sc_agg_disease(x_gene, src_gd, dst_gd)
    g1 = _transform(N_G, acc_g1, None, cnt_g, x_gene,
                    W_l_dg1, W_r_dg1, b_dg1.reshape(1, D))
    d1 = _transform(N_D, acc_d1, True, cnt_d, x_disease,
                    W_l_gd1, W_r_gd1, b_gd1.reshape(1, D))

    acc_g2 = _sc_agg_gene(d1, src_dg, dst_dg)
    acc_d2 = _sc_agg_disease(g1, src_gd, dst_gd)
    g2 = _transform(N_G, acc_g2, None, cnt_g, g1,
                    W_l_dg2, W_r_dg2, b_dg2.reshape(1, D))
    d2 = _transform(N_D, acc_d2, True, cnt_d, d1,
                    W_l_gd2, W_r_gd2, b_gd2.reshape(1, D))
    return (d2, g2)


# speedup vs baseline: 5.2184x; 2.0547x over previous
import functools

import jax
import jax.numpy as jnp
from jax import lax
from jax.experimental import pallas as pl
from jax.experimental.pallas import tpu as pltpu
from jax.experimental.pallas import tpu_sc as plsc

N_D = 10000
N_G = 50000
E = 320000
D = 128

NC = 2
NS = 16

G_CHUNK = 8192
G_NCHUNK = 8
G_PAD = G_CHUNK * G_NCHUNK
G_GARB = 256
G_SH_ROWS = G_CHUNK + G_GARB
G_CNT_PAD = 51200

D_PAD = 10240

W_CNT = 2000
W_PART = 2000
T_EDGE = E // 32
P_CAP = 10240
FLUSH = 1280
B_GENE = 128
B_DIS = 80
W_DIS = 2000

_mesh = plsc.VectorSubcoreMesh(core_axis_name="c", subcore_axis_name="s")

def _zero_vmem_2d(buf, rows):
    z = jnp.zeros((16,), jnp.float32)

    @pl.loop(0, rows)
    def _(r):
        for l in range(8):
            buf[r, pl.ds(l * 16, 16)] = z


def _fill_vmem_1d(buf, n, value):
    v = jnp.full((16,), value, jnp.float32)

    @pl.loop(0, n, step=16)
    def _(i):
        buf[pl.ds(i, 16)] = v


@functools.partial(
    pl.kernel,
    out_type=(
        jax.ShapeDtypeStruct((G_CNT_PAD,), jnp.float32),
        jax.ShapeDtypeStruct((D_PAD,), jnp.float32),
    ),
    mesh=_mesh,
    scratch_types=[
        pltpu.VMEM((3200,), jnp.float32),
        pltpu.VMEM((W_CNT,), jnp.float32),
        pltpu.VMEM((W_CNT,), jnp.int32),
        pltpu.VMEM_SHARED((G_CNT_PAD,), jnp.float32),
        pltpu.VMEM_SHARED((D_PAD,), jnp.float32),
    ],
)
def _sc_counts(dst_dg, dst_gd, cnt_g_out, cnt_d_out, zbuf, ones_v, idx_v,
               sh_g, sh_d):
    core = lax.axis_index("c")
    sid = lax.axis_index("s")
    _fill_vmem_1d(zbuf, 3200, 0.0)
    _fill_vmem_1d(ones_v, W_CNT, 1.0)

    @pl.when(core == 0)
    def _():
        pltpu.sync_copy(zbuf.at[pl.ds(0, G_CNT_PAD // NS)],
                        sh_g.at[pl.ds(sid * (G_CNT_PAD // NS), G_CNT_PAD // NS)])

    @pl.when(core == 1)
    def _():
        pltpu.sync_copy(zbuf.at[pl.ds(0, D_PAD // NS)],
                        sh_d.at[pl.ds(sid * (D_PAD // NS), D_PAD // NS)])

    plsc.subcore_barrier()

    per_tile = E // NS

    @pl.loop(0, per_tile // W_CNT)
    def _(w):
        base = sid * per_tile + w * W_CNT

        @pl.when(core == 0)
        def _():
            pltpu.sync_copy(dst_dg.at[pl.ds(base, W_CNT)], idx_v)
            pltpu.sync_copy(ones_v, sh_g.at[idx_v], add=True)

        @pl.when(core == 1)
        def _():
            pltpu.sync_copy(dst_gd.at[pl.ds(base, W_CNT)], idx_v)
            pltpu.sync_copy(ones_v, sh_d.at[idx_v], add=True)

    plsc.subcore_barrier()

    @pl.when(core == 0)
    def _():
        pltpu.sync_copy(sh_g.at[pl.ds(sid * (G_CNT_PAD // NS), G_CNT_PAD // NS)],
                        cnt_g_out.at[pl.ds(sid * (G_CNT_PAD // NS), G_CNT_PAD // NS)])

    @pl.when(core == 1)
    def _():
        pltpu.sync_copy(sh_d.at[pl.ds(sid * (D_PAD // NS), D_PAD // NS)],
                        cnt_d_out.at[pl.ds(sid * (D_PAD // NS), D_PAD // NS)])


@functools.partial(
    pl.kernel,
    out_type=(
        jax.ShapeDtypeStruct((32 * G_NCHUNK * P_CAP,), jnp.int32),
        jax.ShapeDtypeStruct((32 * G_NCHUNK * P_CAP,), jnp.int32),
        jax.ShapeDtypeStruct((32 * 16,), jnp.int32),
    ),
    compiler_params=pltpu.CompilerParams(needs_layout_passes=False),
    mesh=_mesh,
    scratch_types=[
        pltpu.VMEM((W_PART,), jnp.int32),
        pltpu.VMEM((W_PART,), jnp.int32),
        pltpu.VMEM((G_NCHUNK * (FLUSH + 16),), jnp.int32),
        pltpu.VMEM((G_NCHUNK * (FLUSH + 16),), jnp.int32),
        pltpu.VMEM((16,), jnp.int32),
    ],
)
def _sc_partition_gene(src_idx, dst_idx, psrc_out, pdst_out, ncnt_out,
                       src_v, dst_v, bsrc, bdst, cnt_v):
    core = lax.axis_index("c")
    sid = lax.axis_index("s")
    wid = sid * NC + core
    lane = lax.iota(jnp.int32, 16)
    ebase = wid * T_EDGE

    def _mkflush(c):
        vbase = c * (FLUSH + 16)

        def flush(off, nf):
            hbase = (wid * G_NCHUNK + c) * P_CAP
            pltpu.sync_copy(bsrc.at[pl.ds(vbase, FLUSH)],
                            psrc_out.at[pl.ds(hbase + nf * FLUSH, FLUSH)])
            pltpu.sync_copy(bdst.at[pl.ds(vbase, FLUSH)],
                            pdst_out.at[pl.ds(hbase + nf * FLUSH, FLUSH)])
            bsrc[pl.ds(vbase, 16)] = bsrc[pl.ds(vbase + FLUSH, 16)]
            bdst[pl.ds(vbase, 16)] = bdst[pl.ds(vbase + FLUSH, 16)]
            return off - FLUSH, nf + 1

        def keep(off, nf):
            return off, nf

        return flush, keep

    def vreg(j, carry):
        offs, nfs = carry
        d16 = dst_v[pl.ds(j * 16, 16)]
        s16 = src_v[pl.ds(j * 16, 16)]
        cid = lax.shift_right_logical(d16, 13)
        rel = jnp.bitwise_and(d16, G_CHUNK - 1)
        new_offs, new_nfs = [], []
        for c in range(G_NCHUNK):
            m = cid == c
            off, nf = offs[c], nfs[c]
            vbase = c * (FLUSH + 16)
            plsc.store_compressed(bdst.at[pl.ds(vbase + off, 16)], rel, mask=m)
            plsc.store_compressed(bsrc.at[pl.ds(vbase + off, 16)], s16, mask=m)
            off = off + jnp.max(plsc.all_reduce_population_count(m))
            flush, keep = _mkflush(c)
            off, nf = lax.cond(off >= FLUSH, flush, keep, off, nf)
            new_offs.append(off)
            new_nfs.append(nf)
        return tuple(new_offs), tuple(new_nfs)

    def window(w, carry):
        pltpu.sync_copy(src_idx.at[pl.ds(ebase + w * W_PART, W_PART)], src_v)
        pltpu.sync_copy(dst_idx.at[pl.ds(ebase + w * W_PART, W_PART)], dst_v)
        return lax.fori_loop(0, W_PART // 16, vreg, carry)

    zero = jnp.int32(0)
    offs, nfs = lax.fori_loop(0, T_EDGE // W_PART, window,
                              ((zero,) * G_NCHUNK, (zero,) * G_NCHUNK))

    cnt16 = jnp.zeros((16,), jnp.int32)
    for c in range(G_NCHUNK):
        off, nf = offs[c], nfs[c]
        vbase = c * (FLUSH + 16)
        hbase = (wid * G_NCHUNK + c) * P_CAP
        for k in range(FLUSH // B_GENE):
            @pl.when(k * B_GENE < off)
            def _():
                pltpu.sync_copy(
                    bsrc.at[pl.ds(vbase + k * B_GENE, B_GENE)],
                    psrc_out.at[pl.ds(hbase + nf * FLUSH + k * B_GENE,
                                      B_GENE)])
                pltpu.sync_copy(
                    bdst.at[pl.ds(vbase + k * B_GENE, B_GENE)],
                    pdst_out.at[pl.ds(hbase + nf * FLUSH + k * B_GENE,
                                      B_GENE)])

        cnt16 = jnp.where(lane == c, nf * FLUSH + off, cnt16)

    cnt_v[pl.ds(0, 16)] = cnt16
    pltpu.sync_copy(cnt_v, ncnt_out.at[pl.ds(wid * 16, 16)])


@functools.partial(
    pl.kernel,
    out_type=jax.ShapeDtypeStruct((G_PAD, D), jnp.float32),
    compiler_params=pltpu.CompilerParams(needs_layout_passes=False),
    mesh=_mesh,
    scratch_types=[
        pltpu.VMEM((32, D), jnp.float32),
        pltpu.VMEM((B_GENE,), jnp.int32),
        pltpu.VMEM((B_GENE,), jnp.int32),
        pltpu.VMEM((B_GENE,), jnp.int32),
        pltpu.VMEM((B_GENE,), jnp.int32),
        pltpu.VMEM((B_GENE,), jnp.int32),
        pltpu.VMEM((B_GENE,), jnp.int32),
        pltpu.VMEM((B_GENE,), jnp.int32),
        pltpu.VMEM((B_GENE,), jnp.int32),
        pltpu.VMEM((B_GENE, D), jnp.float32),
        pltpu.VMEM((B_GENE, D), jnp.float32),
        pltpu.VMEM((16,), jnp.int32),
        pltpu.VMEM_SHARED((G_SH_ROWS, D), jnp.float32),
        pltpu.SemaphoreType.DMA,
        pltpu.SemaphoreType.DMA,
    ],
)
def _sc_agg_gene(x_src, psrc, pdst, ncnt, zeros, acc_out,
                 zbuf, sr0, dr0, sr1, dr1, sf0, df0, sf1, df1,
                 rows0, rows1, cnt_v, sh_acc, sem0, sem1):
    core = lax.axis_index("c")
    sid = lax.axis_index("s")
    lane = lax.iota(jnp.int32, 16)
    pltpu.sync_copy(zeros, zbuf)

    zrows = G_SH_ROWS // NS
    orows = G_CHUNK // NS
    raw = ((sr0, dr0), (sr1, dr1))
    fix = ((sf0, df0), (sf1, df1))
    rows = (rows0, rows1)
    sems = (sem0, sem1)

    for ci in range(G_NCHUNK // NC):
        chunk = core + NC * ci

        @pl.loop(0, 512, step=32)
        def _(r):
            pltpu.sync_copy(zbuf.at[pl.ds(0, 32)],
                            sh_acc.at[pl.ds(sid * zrows + r, 32)])

        pltpu.sync_copy(zbuf.at[pl.ds(0, 16)],
                        sh_acc.at[pl.ds(sid * zrows + 512, 16)])

        plsc.subcore_barrier()

        for b in (2 * sid, 2 * sid + 1):
            pltpu.sync_copy(ncnt.at[pl.ds(b * 16, 16)], cnt_v)
            cnt = jnp.max(jnp.where(lane == chunk, cnt_v[pl.ds(0, 16)], 0))
            nb = lax.shift_right_logical(cnt + (B_GENE - 1), 7)

            seg = (b * G_NCHUNK + chunk) * P_CAP

            def load_fix(blk, p):
                pltpu.sync_copy(psrc.at[pl.ds(seg + blk * B_GENE, B_GENE)],
                                raw[p][0])
                pltpu.sync_copy(pdst.at[pl.ds(seg + blk * B_GENE, B_GENE)],
                                raw[p][1])

                @pl.loop(0, B_GENE // 16)
                def _(j):
                    pos = blk * B_GENE + j * 16 + lane
                    ok = pos < cnt
                    s16 = raw[p][0][pl.ds(j * 16, 16)]
                    d16 = raw[p][1][pl.ds(j * 16, 16)]
                    garb = G_CHUNK + lax.rem((b + j) * 16, G_GARB) + lane
                    fix[p][0][pl.ds(j * 16, 16)] = jnp.where(ok, s16,
                                                             j * 16 + lane)
                    fix[p][1][pl.ds(j * 16, 16)] = jnp.where(ok, d16, garb)

                pltpu.async_copy(x_src.at[fix[p][0]], rows[p], sems[p])

            def drain(p):
                pltpu.make_async_copy(x_src.at[fix[p][0]], rows[p],
                                      sems[p]).wait()
                pltpu.sync_copy(rows[p], sh_acc.at[fix[p][1]], add=True)

            @pl.loop(0, lax.shift_right_logical(nb + 1, 1))
            def _(s2):
                blk1 = 2 * s2 + 1
                load_fix(2 * s2, 0)

                @pl.when(blk1 < nb)
                def _():
                    load_fix(blk1, 1)

                drain(0)

                @pl.when(blk1 < nb)
                def _():
                    drain(1)

        plsc.subcore_barrier()
        pltpu.sync_copy(sh_acc.at[pl.ds(sid * orows, orows)],
                        acc_out.at[pl.ds(chunk * G_CHUNK + sid * orows,
                                         orows)])
        plsc.subcore_barrier()


@functools.partial(
    pl.kernel,
    out_type=jax.ShapeDtypeStruct((NC * D_PAD, D), jnp.float32),
    mesh=_mesh,
    scratch_types=[
        pltpu.VMEM((32, D), jnp.float32),
        pltpu.VMEM((W_DIS,), jnp.int32),
        pltpu.VMEM((W_DIS,), jnp.int32),
        pltpu.VMEM((B_DIS,), jnp.int32),
        pltpu.VMEM((B_DIS,), jnp.int32),
        pltpu.VMEM((B_DIS, D), jnp.float32),
        pltpu.VMEM((B_DIS, D), jnp.float32),
        pltpu.VMEM_SHARED((D_PAD, D), jnp.float32),
        pltpu.SemaphoreType.DMA,
        pltpu.SemaphoreType.DMA,
    ],
)
def _sc_agg_disease(x_src, src_idx, dst_idx, acc_out, zbuf, src_v, dst_v,
                    df0, df1, rows0, rows1, sh_acc, sem0, sem1):
    core = lax.axis_index("c")
    sid = lax.axis_index("s")
    _zero_vmem_2d(zbuf, 32)

    per_tile = E // (NC * NS)
    zrows = D_PAD // NS
    dfix = (df0, df1)
    rows = (rows0, rows1)
    sems = (sem0, sem1)

    @pl.loop(0, zrows, step=32)
    def _(r):
        pltpu.sync_copy(zbuf.at[pl.ds(0, 32)],
                        sh_acc.at[pl.ds(sid * zrows + r, 32)])

    plsc.subcore_barrier()

    @pl.loop(0, per_tile // W_DIS)
    def _(w):
        base = core * (E // NC) + sid * per_tile + w * W_DIS
        pltpu.sync_copy(src_idx.at[pl.ds(base, W_DIS)], src_v)
        pltpu.sync_copy(dst_idx.at[pl.ds(base, W_DIS)], dst_v)

        def start(s, p):
            @pl.loop(0, B_DIS // 16)
            def _(j):
                dfix[p][pl.ds(j * 16, 16)] = dst_v[pl.ds(s * B_DIS + j * 16,
                                                         16)]

            pltpu.async_copy(x_src.at[src_v.at[pl.ds(s * B_DIS, B_DIS)]],
                             rows[p], sems[p])

        def drain(s, p):
            pltpu.make_async_copy(
                x_src.at[src_v.at[pl.ds(s * B_DIS, B_DIS)]],
                rows[p], sems[p]).wait()
            pltpu.sync_copy(rows[p], sh_acc.at[dfix[p]], add=True)

        nblk = W_DIS // B_DIS

        @pl.loop(0, nblk // 2)
        def _(s2):
            start(2 * s2, 0)
            start(2 * s2 + 1, 1)
            drain(2 * s2, 0)
            drain(2 * s2 + 1, 1)

        start(nblk - 1, 0)
        drain(nblk - 1, 0)

    plsc.subcore_barrier()
    pltpu.sync_copy(sh_acc.at[pl.ds(sid * zrows, zrows)],
                    acc_out.at[pl.ds(core * D_PAD + sid * zrows, zrows)])


def _transform_body(two_acc, a0_ref, a1_ref, cnt_ref, x_ref, wl_ref, wr_ref,
                    b_ref, o_ref):
    if two_acc:
        agg = a0_ref[...] + a1_ref[...]
    else:
        agg = a0_ref[...]
        del a1_ref
    inv = 1.0 / jnp.maximum(cnt_ref[...], 1.0)
    agg = agg * inv
    dn = (((1,), (1,)), ((), ()))
    out = lax.dot_general(agg, wl_ref[...], dn,
                          precision=lax.Precision.HIGHEST,
                          preferred_element_type=jnp.float32)
    out += lax.dot_general(x_ref[...], wr_ref[...], dn,
                           precision=lax.Precision.HIGHEST,
                           preferred_element_type=jnp.float32)
    o_ref[...] = out + b_ref[...]


def _transform(n_rows, acc, partials, cnt, x_dst, w_l, w_r, b):
    if partials:
        bk = 1280
        grid = ((n_rows + bk - 1) // bk,)
        nblk = D_PAD // bk
        in_specs = [
            pl.BlockSpec((bk, D), lambda i: (i, 0)),
            pl.BlockSpec((bk, D), lambda i: (i + nblk, 0)),
        ]
        args = (acc, acc)
        body = functools.partial(_transform_body, True)
    else:
        bk = 2000
        grid = (n_rows // bk,)
        in_specs = [
            pl.BlockSpec((bk, D), lambda i: (i, 0)),
        ]
        args = (acc,)

        def body(a0, cnt_r, x_r, wl, wr, b_r, o_r):
            _transform_body(False, a0, None, cnt_r, x_r, wl, wr, b_r, o_r)
    in_specs += [
        pl.BlockSpec((bk, 1), lambda i: (i, 0)),
        pl.BlockSpec((bk, D), lambda i: (i, 0)),
        pl.BlockSpec((D, D), lambda i: (0, 0)),
        pl.BlockSpec((D, D), lambda i: (0, 0)),
        pl.BlockSpec((1, D), lambda i: (0, 0)),
    ]
    return pl.pallas_call(
        body,
        grid=grid,
        in_specs=in_specs,
        out_specs=pl.BlockSpec((bk, D), lambda i: (i, 0)),
        out_shape=jax.ShapeDtypeStruct((n_rows, D), jnp.float32),
    )(*args, cnt, x_dst, w_l, w_r, b)


def kernel(x_disease, x_gene, src_dg, dst_dg, src_gd, dst_gd,
           W_l_dg1, W_r_dg1, W_l_gd1, W_r_gd1,
           W_l_dg2, W_r_dg2, W_l_gd2, W_r_gd2,
           b_dg1, b_gd1, b_dg2, b_gd2):
    cnt_g, cnt_d = _sc_counts(dst_dg, dst_gd)
    cnt_g = cnt_g.reshape(G_CNT_PAD, 1)
    cnt_d = cnt_d.reshape(D_PAD, 1)
    psrc, pdst, ncnt = _sc_partition_gene(src_dg, dst_dg)

    zeros = jnp.zeros((32, D), jnp.float32)
    acc_g1 = _sc_agg_gene(x_disease, psrc, pdst, ncnt, zeros)
    acc_d1 = _sc_agg_disease(x_gene, src_gd, dst_gd)
    g1 = _transform(N_G, acc_g1, None, cnt_g, x_gene,
                    W_l_dg1, W_r_dg1, b_dg1.reshape(1, D))
    d1 = _transform(N_D, acc_d1, True, cnt_d, x_disease,
                    W_l_gd1, W_r_gd1, b_gd1.reshape(1, D))

    acc_g2 = _sc_agg_gene(d1, psrc, pdst, ncnt, zeros)
    acc_d2 = _sc_agg_disease(g1, src_gd, dst_gd)
    g2 = _transform(N_G, acc_g2, None, cnt_g, g1,
                    W_l_dg2, W_r_dg2, b_dg2.reshape(1, D))
    d2 = _transform(N_D, acc_d2, True, cnt_d, d1,
                    W_l_gd2, W_r_gd2, b_gd2.reshape(1, D))
    return (d2, g2)


# speedup vs baseline: 5.7056x; 1.0934x over previous
import functools

import jax
import jax.numpy as jnp
from jax import lax
from jax.experimental import pallas as pl
from jax.experimental.pallas import tpu as pltpu
from jax.experimental.pallas import tpu_sc as plsc

N_D = 10000
N_G = 50000
E = 320000
D = 128

NC = 2
NS = 16

G_CHUNK = 8192
G_NCHUNK = 8
G_PAD = G_CHUNK * G_NCHUNK
G_GARB = 256
G_SH_ROWS = G_CHUNK + G_GARB
G_CNT_PAD = 51200

D_PAD = 10240

W_CNT = 2000
W_PART = 2000
T_EDGE = E // 32
P_CAP = 10240
FLUSH = 1280
B_GENE = 128
B_DIS = 80
W_DIS = 2000

_mesh = plsc.VectorSubcoreMesh(core_axis_name="c", subcore_axis_name="s")

def _zero_vmem_2d(buf, rows):
    z = jnp.zeros((16,), jnp.float32)

    @pl.loop(0, rows)
    def _(r):
        for l in range(8):
            buf[r, pl.ds(l * 16, 16)] = z


def _fill_vmem_1d(buf, n, value):
    v = jnp.full((16,), value, jnp.float32)

    @pl.loop(0, n, step=16)
    def _(i):
        buf[pl.ds(i, 16)] = v


@functools.partial(
    pl.kernel,
    out_type=(
        jax.ShapeDtypeStruct((G_CNT_PAD,), jnp.float32),
        jax.ShapeDtypeStruct((D_PAD,), jnp.float32),
    ),
    mesh=_mesh,
    scratch_types=[
        pltpu.VMEM((3200,), jnp.float32),
        pltpu.VMEM((W_CNT,), jnp.float32),
        pltpu.VMEM((W_CNT,), jnp.int32),
        pltpu.VMEM_SHARED((G_CNT_PAD,), jnp.float32),
        pltpu.VMEM_SHARED((D_PAD,), jnp.float32),
    ],
)
def _sc_counts(dst_dg, dst_gd, cnt_g_out, cnt_d_out, zbuf, ones_v, idx_v,
               sh_g, sh_d):
    core = lax.axis_index("c")
    sid = lax.axis_index("s")
    _fill_vmem_1d(zbuf, 3200, 0.0)
    _fill_vmem_1d(ones_v, W_CNT, 1.0)

    @pl.when(core == 0)
    def _():
        pltpu.sync_copy(zbuf.at[pl.ds(0, G_CNT_PAD // NS)],
                        sh_g.at[pl.ds(sid * (G_CNT_PAD // NS), G_CNT_PAD // NS)])

    @pl.when(core == 1)
    def _():
        pltpu.sync_copy(zbuf.at[pl.ds(0, D_PAD // NS)],
                        sh_d.at[pl.ds(sid * (D_PAD // NS), D_PAD // NS)])

    plsc.subcore_barrier()

    per_tile = E // NS

    @pl.loop(0, per_tile // W_CNT)
    def _(w):
        base = sid * per_tile + w * W_CNT

        @pl.when(core == 0)
        def _():
            pltpu.sync_copy(dst_dg.at[pl.ds(base, W_CNT)], idx_v)
            pltpu.sync_copy(ones_v, sh_g.at[idx_v], add=True)

        @pl.when(core == 1)
        def _():
            pltpu.sync_copy(dst_gd.at[pl.ds(base, W_CNT)], idx_v)
            pltpu.sync_copy(ones_v, sh_d.at[idx_v], add=True)

    plsc.subcore_barrier()

    @pl.when(core == 0)
    def _():
        pltpu.sync_copy(sh_g.at[pl.ds(sid * (G_CNT_PAD // NS), G_CNT_PAD // NS)],
                        cnt_g_out.at[pl.ds(sid * (G_CNT_PAD // NS), G_CNT_PAD // NS)])

    @pl.when(core == 1)
    def _():
        pltpu.sync_copy(sh_d.at[pl.ds(sid * (D_PAD // NS), D_PAD // NS)],
                        cnt_d_out.at[pl.ds(sid * (D_PAD // NS), D_PAD // NS)])


@functools.partial(
    pl.kernel,
    out_type=(
        jax.ShapeDtypeStruct((32 * G_NCHUNK * P_CAP,), jnp.int32),
        jax.ShapeDtypeStruct((32 * G_NCHUNK * P_CAP,), jnp.int32),
        jax.ShapeDtypeStruct((32 * 16,), jnp.int32),
    ),
    compiler_params=pltpu.CompilerParams(needs_layout_passes=False),
    mesh=_mesh,
    scratch_types=[
        pltpu.VMEM((W_PART,), jnp.int32),
        pltpu.VMEM((W_PART,), jnp.int32),
        pltpu.VMEM((G_NCHUNK * (FLUSH + 16),), jnp.int32),
        pltpu.VMEM((G_NCHUNK * (FLUSH + 16),), jnp.int32),
        pltpu.VMEM((16,), jnp.int32),
    ],
)
def _sc_partition_gene(src_idx, dst_idx, psrc_out, pdst_out, ncnt_out,
                       src_v, dst_v, bsrc, bdst, cnt_v):
    core = lax.axis_index("c")
    sid = lax.axis_index("s")
    wid = sid * NC + core
    lane = lax.iota(jnp.int32, 16)
    ebase = wid * T_EDGE

    def _mkflush(c):
        vbase = c * (FLUSH + 16)

        def flush(off, nf):
            hbase = (wid * G_NCHUNK + c) * P_CAP
            pltpu.sync_copy(bsrc.at[pl.ds(vbase, FLUSH)],
                            psrc_out.at[pl.ds(hbase + nf * FLUSH, FLUSH)])
            pltpu.sync_copy(bdst.at[pl.ds(vbase, FLUSH)],
                            pdst_out.at[pl.ds(hbase + nf * FLUSH, FLUSH)])
            bsrc[pl.ds(vbase, 16)] = bsrc[pl.ds(vbase + FLUSH, 16)]
            bdst[pl.ds(vbase, 16)] = bdst[pl.ds(vbase + FLUSH, 16)]
            return off - FLUSH, nf + 1

        def keep(off, nf):
            return off, nf

        return flush, keep

    def vreg(j, carry):
        offs, nfs = carry
        d16 = dst_v[pl.ds(j * 16, 16)]
        s16 = src_v[pl.ds(j * 16, 16)]
        cid = lax.shift_right_logical(d16, 13)
        rel = jnp.bitwise_and(d16, G_CHUNK - 1)
        new_offs, new_nfs = [], []
        for c in range(G_NCHUNK):
            m = cid == c
            off, nf = offs[c], nfs[c]
            vbase = c * (FLUSH + 16)
            plsc.store_compressed(bdst.at[pl.ds(vbase + off, 16)], rel, mask=m)
            plsc.store_compressed(bsrc.at[pl.ds(vbase + off, 16)], s16, mask=m)
            off = off + jnp.max(plsc.all_reduce_population_count(m))
            flush, keep = _mkflush(c)
            off, nf = lax.cond(off >= FLUSH, flush, keep, off, nf)
            new_offs.append(off)
            new_nfs.append(nf)
        return tuple(new_offs), tuple(new_nfs)

    def window(w, carry):
        pltpu.sync_copy(src_idx.at[pl.ds(ebase + w * W_PART, W_PART)], src_v)
        pltpu.sync_copy(dst_idx.at[pl.ds(ebase + w * W_PART, W_PART)], dst_v)
        return lax.fori_loop(0, W_PART // 16, vreg, carry)

    zero = jnp.int32(0)
    offs, nfs = lax.fori_loop(0, T_EDGE // W_PART, window,
                              ((zero,) * G_NCHUNK, (zero,) * G_NCHUNK))

    cnt16 = jnp.zeros((16,), jnp.int32)
    for c in range(G_NCHUNK):
        off, nf = offs[c], nfs[c]
        vbase = c * (FLUSH + 16)
        hbase = (wid * G_NCHUNK + c) * P_CAP
        for k in range(FLUSH // B_GENE):
            @pl.when(k * B_GENE < off)
            def _():
                pltpu.sync_copy(
                    bsrc.at[pl.ds(vbase + k * B_GENE, B_GENE)],
                    psrc_out.at[pl.ds(hbase + nf * FLUSH + k * B_GENE,
                                      B_GENE)])
                pltpu.sync_copy(
                    bdst.at[pl.ds(vbase + k * B_GENE, B_GENE)],
                    pdst_out.at[pl.ds(hbase + nf * FLUSH + k * B_GENE,
                                      B_GENE)])

        cnt16 = jnp.where(lane == c, nf * FLUSH + off, cnt16)

    cnt_v[pl.ds(0, 16)] = cnt16
    pltpu.sync_copy(cnt_v, ncnt_out.at[pl.ds(wid * 16, 16)])


@functools.partial(
    pl.kernel,
    out_type=jax.ShapeDtypeStruct((G_PAD, D), jnp.float32),
    compiler_params=pltpu.CompilerParams(needs_layout_passes=False),
    mesh=_mesh,
    scratch_types=[
        pltpu.VMEM((32, D), jnp.float32),
        pltpu.VMEM((FLUSH,), jnp.int32),
        pltpu.VMEM((FLUSH,), jnp.int32),
        pltpu.VMEM((B_GENE,), jnp.int32),
        pltpu.VMEM((B_GENE,), jnp.int32),
        pltpu.VMEM((B_GENE,), jnp.int32),
        pltpu.VMEM((B_GENE,), jnp.int32),
        pltpu.VMEM((B_GENE, D), jnp.float32),
        pltpu.VMEM((B_GENE, D), jnp.float32),
        pltpu.VMEM((16,), jnp.int32),
        pltpu.VMEM_SHARED((G_SH_ROWS, D), jnp.float32),
        pltpu.SemaphoreType.DMA,
        pltpu.SemaphoreType.DMA,
        pltpu.SemaphoreType.DMA,
        pltpu.SemaphoreType.DMA,
    ],
)
def _sc_agg_gene(x_src, psrc, pdst, ncnt, zeros, acc_out,
                 zbuf, swin, dwin, sf0, df0, sf1, df1,
                 rows0, rows1, cnt_v, sh_acc, sem0, sem1, ssem0, ssem1):
    core = lax.axis_index("c")
    sid = lax.axis_index("s")
    lane = lax.iota(jnp.int32, 16)
    pltpu.sync_copy(zeros, zbuf)

    zrows = G_SH_ROWS // NS
    orows = G_CHUNK // NS
    fix = ((sf0, df0), (sf1, df1))
    rows = (rows0, rows1)
    sems = (sem0, sem1)
    ssems = (ssem0, ssem1)
    blk_per_win = FLUSH // B_GENE

    for ci in range(G_NCHUNK // NC):
        chunk = core + NC * ci

        @pl.loop(0, 512, step=32)
        def _(r):
            pltpu.sync_copy(zbuf.at[pl.ds(0, 32)],
                            sh_acc.at[pl.ds(sid * zrows + r, 32)])

        pltpu.sync_copy(zbuf.at[pl.ds(0, 16)],
                        sh_acc.at[pl.ds(sid * zrows + 512, 16)])

        plsc.subcore_barrier()

        for b in (2 * sid, 2 * sid + 1):
            pltpu.sync_copy(ncnt.at[pl.ds(b * 16, 16)], cnt_v)
            cnt = jnp.max(jnp.where(lane == chunk, cnt_v[pl.ds(0, 16)], 0))
            nb = lax.shift_right_logical(cnt + (B_GENE - 1), 7)

            seg = (b * G_NCHUNK + chunk) * P_CAP

            def fix_start(blk, koff, p):
                @pl.when(blk >= 2)
                def _():
                    pltpu.make_async_copy(rows[p], sh_acc.at[fix[p][1]],
                                          ssems[p]).wait()

                @pl.loop(0, B_GENE // 16)
                def _(j):
                    pos = blk * B_GENE + j * 16 + lane
                    ok = pos < cnt
                    s16 = swin[pl.ds(koff + j * 16, 16)]
                    d16 = dwin[pl.ds(koff + j * 16, 16)]
                    garb = G_CHUNK + lax.rem((b + j) * 16, G_GARB) + lane
                    fix[p][0][pl.ds(j * 16, 16)] = jnp.where(ok, s16,
                                                             j * 16 + lane)
                    fix[p][1][pl.ds(j * 16, 16)] = jnp.where(ok, d16, garb)

                pltpu.async_copy(x_src.at[fix[p][0]], rows[p], sems[p])

            def drain(p):
                pltpu.make_async_copy(x_src.at[fix[p][0]], rows[p],
                                      sems[p]).wait()
                pltpu.async_copy(rows[p], sh_acc.at[fix[p][1]], ssems[p],
                                 add=True)

            @pl.loop(0, P_CAP // FLUSH)
            def _(w):
                @pl.when(w * FLUSH < cnt)
                def _():
                    pltpu.sync_copy(psrc.at[pl.ds(seg + w * FLUSH, FLUSH)],
                                    swin)
                    pltpu.sync_copy(pdst.at[pl.ds(seg + w * FLUSH, FLUSH)],
                                    dwin)

                    @pl.loop(0, blk_per_win // 2)
                    def _(kp):
                        blk0 = w * blk_per_win + 2 * kp
                        blk1 = blk0 + 1

                        @pl.when(blk0 < nb)
                        def _():
                            fix_start(blk0, 2 * kp * B_GENE, 0)

                        @pl.when(blk1 < nb)
                        def _():
                            fix_start(blk1, (2 * kp + 1) * B_GENE, 1)

                        @pl.when(blk0 < nb)
                        def _():
                            drain(0)

                        @pl.when(blk1 < nb)
                        def _():
                            drain(1)

            @pl.when(nb >= 1)
            def _():
                pltpu.make_async_copy(rows[0], sh_acc.at[fix[0][1]],
                                      ssems[0]).wait()

            @pl.when(nb >= 2)
            def _():
                pltpu.make_async_copy(rows[1], sh_acc.at[fix[1][1]],
                                      ssems[1]).wait()

        plsc.subcore_barrier()
        pltpu.sync_copy(sh_acc.at[pl.ds(sid * orows, orows)],
                        acc_out.at[pl.ds(chunk * G_CHUNK + sid * orows,
                                         orows)])
        plsc.subcore_barrier()


T_DIS = E // (NC * NS)
NB_DIS = T_DIS // B_DIS


@functools.partial(
    pl.kernel,
    out_type=jax.ShapeDtypeStruct((NC * D_PAD, D), jnp.float32),
    mesh=_mesh,
    scratch_types=[
        pltpu.VMEM((32, D), jnp.float32),
        pltpu.VMEM((T_DIS,), jnp.int32),
        pltpu.VMEM((T_DIS,), jnp.int32),
        pltpu.VMEM((B_DIS,), jnp.int32),
        pltpu.VMEM((B_DIS,), jnp.int32),
        pltpu.VMEM((B_DIS, D), jnp.float32),
        pltpu.VMEM((B_DIS, D), jnp.float32),
        pltpu.VMEM_SHARED((D_PAD, D), jnp.float32),
        pltpu.SemaphoreType.DMA,
        pltpu.SemaphoreType.DMA,
        pltpu.SemaphoreType.DMA,
        pltpu.SemaphoreType.DMA,
    ],
)
def _sc_agg_disease(x_src, src_idx, dst_idx, acc_out, zbuf, src_v, dst_v,
                    df0, df1, rows0, rows1, sh_acc, sem0, sem1, ssem0, ssem1):
    core = lax.axis_index("c")
    sid = lax.axis_index("s")
    _zero_vmem_2d(zbuf, 32)

    zrows = D_PAD // NS
    dfix = (df0, df1)
    rows = (rows0, rows1)
    sems = (sem0, sem1)
    ssems = (ssem0, ssem1)

    @pl.loop(0, zrows, step=32)
    def _(r):
        pltpu.sync_copy(zbuf.at[pl.ds(0, 32)],
                        sh_acc.at[pl.ds(sid * zrows + r, 32)])

    plsc.subcore_barrier()

    base = core * (E // NC) + sid * T_DIS
    pltpu.sync_copy(src_idx.at[pl.ds(base, T_DIS)], src_v)
    pltpu.sync_copy(dst_idx.at[pl.ds(base, T_DIS)], dst_v)

    def start(s, p, first):
        if not first:
            pltpu.make_async_copy(rows[p], sh_acc.at[dfix[p]],
                                  ssems[p]).wait()

        @pl.loop(0, B_DIS // 16)
        def _(j):
            dfix[p][pl.ds(j * 16, 16)] = dst_v[pl.ds(s * B_DIS + j * 16, 16)]

        pltpu.async_copy(x_src.at[src_v.at[pl.ds(s * B_DIS, B_DIS)]],
                         rows[p], sems[p])

    def drain(s, p):
        pltpu.make_async_copy(
            x_src.at[src_v.at[pl.ds(s * B_DIS, B_DIS)]],
            rows[p], sems[p]).wait()
        pltpu.async_copy(rows[p], sh_acc.at[dfix[p]], ssems[p], add=True)

    start(0, 0, True)
    start(1, 1, True)
    drain(0, 0)
    drain(1, 1)

    @pl.loop(1, NB_DIS // 2)
    def _(s2):
        start(2 * s2, 0, False)
        start(2 * s2 + 1, 1, False)
        drain(2 * s2, 0)
        drain(2 * s2 + 1, 1)

    start(NB_DIS - 1, 0, False)
    drain(NB_DIS - 1, 0)

    pltpu.make_async_copy(rows[0], sh_acc.at[dfix[0]], ssems[0]).wait()
    pltpu.make_async_copy(rows[1], sh_acc.at[dfix[1]], ssems[1]).wait()

    plsc.subcore_barrier()
    pltpu.sync_copy(sh_acc.at[pl.ds(sid * zrows, zrows)],
                    acc_out.at[pl.ds(core * D_PAD + sid * zrows, zrows)])


def _transform_body(two_acc, a0_ref, a1_ref, cnt_ref, x_ref, wl_ref, wr_ref,
                    b_ref, o_ref):
    if two_acc:
        agg = a0_ref[...] + a1_ref[...]
    else:
        agg = a0_ref[...]
        del a1_ref
    inv = 1.0 / jnp.maximum(cnt_ref[...], 1.0)
    agg = agg * inv
    dn = (((1,), (1,)), ((), ()))
    out = lax.dot_general(agg, wl_ref[...], dn,
                          precision=lax.Precision.HIGHEST,
                          preferred_element_type=jnp.float32)
    out += lax.dot_general(x_ref[...], wr_ref[...], dn,
                           precision=lax.Precision.HIGHEST,
                           preferred_element_type=jnp.float32)
    o_ref[...] = out + b_ref[...]


def _transform(n_rows, acc, partials, cnt, x_dst, w_l, w_r, b):
    if partials:
        bk = 1280
        grid = ((n_rows + bk - 1) // bk,)
        nblk = D_PAD // bk
        in_specs = [
            pl.BlockSpec((bk, D), lambda i: (i, 0)),
            pl.BlockSpec((bk, D), lambda i: (i + nblk, 0)),
        ]
        args = (acc, acc)
        body = functools.partial(_transform_body, True)
    else:
        bk = 2000
        grid = (n_rows // bk,)
        in_specs = [
            pl.BlockSpec((bk, D), lambda i: (i, 0)),
        ]
        args = (acc,)

        def body(a0, cnt_r, x_r, wl, wr, b_r, o_r):
            _transform_body(False, a0, None, cnt_r, x_r, wl, wr, b_r, o_r)
    in_specs += [
        pl.BlockSpec((bk, 1), lambda i: (i, 0)),
        pl.BlockSpec((bk, D), lambda i: (i, 0)),
        pl.BlockSpec((D, D), lambda i: (0, 0)),
        pl.BlockSpec((D, D), lambda i: (0, 0)),
        pl.BlockSpec((1, D), lambda i: (0, 0)),
    ]
    return pl.pallas_call(
        body,
        grid=grid,
        in_specs=in_specs,
        out_specs=pl.BlockSpec((bk, D), lambda i: (i, 0)),
        out_shape=jax.ShapeDtypeStruct((n_rows, D), jnp.float32),
    )(*args, cnt, x_dst, w_l, w_r, b)


def kernel(x_disease, x_gene, src_dg, dst_dg, src_gd, dst_gd,
           W_l_dg1, W_r_dg1, W_l_gd1, W_r_gd1,
           W_l_dg2, W_r_dg2, W_l_gd2, W_r_gd2,
           b_dg1, b_gd1, b_dg2, b_gd2):
    cnt_g, cnt_d = _sc_counts(dst_dg, dst_gd)
    cnt_g = cnt_g.reshape(G_CNT_PAD, 1)
    cnt_d = cnt_d.reshape(D_PAD, 1)
    psrc, pdst, ncnt = _sc_partition_gene(src_dg, dst_dg)

    zeros = jnp.zeros((32, D), jnp.float32)
    acc_g1 = _sc_agg_gene(x_disease, psrc, pdst, ncnt, zeros)
    acc_d1 = _sc_agg_disease(x_gene, src_gd, dst_gd)
    g1 = _transform(N_G, acc_g1, None, cnt_g, x_gene,
                    W_l_dg1, W_r_dg1, b_dg1.reshape(1, D))
    d1 = _transform(N_D, acc_d1, True, cnt_d, x_disease,
                    W_l_gd1, W_r_gd1, b_gd1.reshape(1, D))

    acc_g2 = _sc_agg_gene(d1, psrc, pdst, ncnt, zeros)
    acc_d2 = _sc_agg_disease(g1, src_gd, dst_gd)
    g2 = _transform(N_G, acc_g2, None, cnt_g, g1,
                    W_l_dg2, W_r_dg2, b_dg2.reshape(1, D))
    d2 = _transform(N_D, acc_d2, True, cnt_d, d1,
                    W_l_gd2, W_r_gd2, b_gd2.reshape(1, D))
    return (d2, g2)


# speedup vs baseline: 5.7583x; 1.0093x over previous
import functools

import jax
import jax.numpy as jnp
from jax import lax
from jax.experimental import pallas as pl
from jax.experimental.pallas import tpu as pltpu
from jax.experimental.pallas import tpu_sc as plsc

N_D = 10000
N_G = 50000
E = 320000
D = 128

NC = 2
NS = 16

G_CHUNK = 8192
G_NCHUNK = 8
G_PAD = G_CHUNK * G_NCHUNK
G_GARB = 256
G_SH_ROWS = G_CHUNK + G_GARB
G_CNT_PAD = 51200

D_CHUNK = 5120
D_NCHUNK = 2
D_PAD = D_CHUNK * D_NCHUNK
D_SH_ROWS = D_CHUNK + 256

W_CNT = 2000
W_PART = 2000
T_EDGE = E // 32
P_CAP = 10240
FLUSH = 2560
FLUSH_D = 2048
B_GENE = 160
B_DIS = 256

_mesh = plsc.VectorSubcoreMesh(core_axis_name="c", subcore_axis_name="s")

def _zero_vmem_2d(buf, rows):
    z = jnp.zeros((16,), jnp.float32)

    @pl.loop(0, rows)
    def _(r):
        for l in range(8):
            buf[r, pl.ds(l * 16, 16)] = z


def _fill_vmem_1d(buf, n, value):
    v = jnp.full((16,), value, jnp.float32)

    @pl.loop(0, n, step=16)
    def _(i):
        buf[pl.ds(i, 16)] = v


@functools.partial(
    pl.kernel,
    out_type=(
        jax.ShapeDtypeStruct((G_CNT_PAD,), jnp.float32),
        jax.ShapeDtypeStruct((D_PAD,), jnp.float32),
    ),
    mesh=_mesh,
    scratch_types=[
        pltpu.VMEM((3200,), jnp.float32),
        pltpu.VMEM((W_CNT,), jnp.float32),
        pltpu.VMEM((W_CNT,), jnp.int32),
        pltpu.VMEM_SHARED((G_CNT_PAD,), jnp.float32),
        pltpu.VMEM_SHARED((D_PAD,), jnp.float32),
    ],
)
def _sc_counts(dst_dg, dst_gd, cnt_g_out, cnt_d_out, zbuf, ones_v, idx_v,
               sh_g, sh_d):
    core = lax.axis_index("c")
    sid = lax.axis_index("s")
    _fill_vmem_1d(zbuf, 3200, 0.0)
    _fill_vmem_1d(ones_v, W_CNT, 1.0)

    @pl.when(core == 0)
    def _():
        pltpu.sync_copy(zbuf.at[pl.ds(0, G_CNT_PAD // NS)],
                        sh_g.at[pl.ds(sid * (G_CNT_PAD // NS), G_CNT_PAD // NS)])

    @pl.when(core == 1)
    def _():
        pltpu.sync_copy(zbuf.at[pl.ds(0, D_PAD // NS)],
                        sh_d.at[pl.ds(sid * (D_PAD // NS), D_PAD // NS)])

    plsc.subcore_barrier()

    per_tile = E // NS

    @pl.loop(0, per_tile // W_CNT)
    def _(w):
        base = sid * per_tile + w * W_CNT

        @pl.when(core == 0)
        def _():
            pltpu.sync_copy(dst_dg.at[pl.ds(base, W_CNT)], idx_v)
            pltpu.sync_copy(ones_v, sh_g.at[idx_v], add=True)

        @pl.when(core == 1)
        def _():
            pltpu.sync_copy(dst_gd.at[pl.ds(base, W_CNT)], idx_v)
            pltpu.sync_copy(ones_v, sh_d.at[idx_v], add=True)

    plsc.subcore_barrier()

    @pl.when(core == 0)
    def _():
        pltpu.sync_copy(sh_g.at[pl.ds(sid * (G_CNT_PAD // NS), G_CNT_PAD // NS)],
                        cnt_g_out.at[pl.ds(sid * (G_CNT_PAD // NS), G_CNT_PAD // NS)])

    @pl.when(core == 1)
    def _():
        pltpu.sync_copy(sh_d.at[pl.ds(sid * (D_PAD // NS), D_PAD // NS)],
                        cnt_d_out.at[pl.ds(sid * (D_PAD // NS), D_PAD // NS)])


def _make_partition(nch, ch_rows, fl, b_blk):
    @functools.partial(
        pl.kernel,
        out_type=(
            jax.ShapeDtypeStruct((32 * nch * P_CAP,), jnp.int32),
            jax.ShapeDtypeStruct((32 * nch * P_CAP,), jnp.int32),
            jax.ShapeDtypeStruct((32 * 16,), jnp.int32),
        ),
        compiler_params=pltpu.CompilerParams(needs_layout_passes=False),
        mesh=_mesh,
        scratch_types=[
            pltpu.VMEM((W_PART,), jnp.int32),
            pltpu.VMEM((W_PART,), jnp.int32),
            pltpu.VMEM((nch * (fl + 16),), jnp.int32),
            pltpu.VMEM((nch * (fl + 16),), jnp.int32),
            pltpu.VMEM((16,), jnp.int32),
        ],
    )
    def part(src_idx, dst_idx, psrc_out, pdst_out, ncnt_out,
             src_v, dst_v, bsrc, bdst, cnt_v):
        core = lax.axis_index("c")
        sid = lax.axis_index("s")
        wid = sid * NC + core
        lane = lax.iota(jnp.int32, 16)
        ebase = wid * T_EDGE

        def _mkflush(c):
            vbase = c * (fl + 16)

            def flush(off, nf):
                hbase = (wid * nch + c) * P_CAP
                pltpu.sync_copy(bsrc.at[pl.ds(vbase, fl)],
                                psrc_out.at[pl.ds(hbase + nf * fl, fl)])
                pltpu.sync_copy(bdst.at[pl.ds(vbase, fl)],
                                pdst_out.at[pl.ds(hbase + nf * fl, fl)])
                bsrc[pl.ds(vbase, 16)] = bsrc[pl.ds(vbase + fl, 16)]
                bdst[pl.ds(vbase, 16)] = bdst[pl.ds(vbase + fl, 16)]
                return off - fl, nf + 1

            def keep(off, nf):
                return off, nf

            return flush, keep

        def vreg(j, carry):
            offs, nfs = carry
            d16 = dst_v[pl.ds(j * 16, 16)]
            s16 = src_v[pl.ds(j * 16, 16)]
            new_offs, new_nfs = [], []
            for c in range(nch):
                lo = c * ch_rows
                m = (d16 >= lo) & (d16 < lo + ch_rows)
                off, nf = offs[c], nfs[c]
                vbase = c * (fl + 16)
                plsc.store_compressed(bdst.at[pl.ds(vbase + off, 16)],
                                      d16 - lo, mask=m)
                plsc.store_compressed(bsrc.at[pl.ds(vbase + off, 16)],
                                      s16, mask=m)
                off = off + jnp.max(plsc.all_reduce_population_count(m))
                flush, keep = _mkflush(c)
                off, nf = lax.cond(off >= fl, flush, keep, off, nf)
                new_offs.append(off)
                new_nfs.append(nf)
            return tuple(new_offs), tuple(new_nfs)

        def window(w, carry):
            pltpu.sync_copy(src_idx.at[pl.ds(ebase + w * W_PART, W_PART)],
                            src_v)
            pltpu.sync_copy(dst_idx.at[pl.ds(ebase + w * W_PART, W_PART)],
                            dst_v)
            return lax.fori_loop(0, W_PART // 16, vreg, carry)

        zero = jnp.int32(0)
        offs, nfs = lax.fori_loop(0, T_EDGE // W_PART, window,
                                  ((zero,) * nch, (zero,) * nch))

        cnt16 = jnp.zeros((16,), jnp.int32)
        for c in range(nch):
            off, nf = offs[c], nfs[c]
            vbase = c * (fl + 16)
            hbase = (wid * nch + c) * P_CAP
            for k in range(fl // b_blk):
                @pl.when(k * b_blk < off)
                def _():
                    pltpu.sync_copy(
                        bsrc.at[pl.ds(vbase + k * b_blk, b_blk)],
                        psrc_out.at[pl.ds(hbase + nf * fl + k * b_blk,
                                          b_blk)])
                    pltpu.sync_copy(
                        bdst.at[pl.ds(vbase + k * b_blk, b_blk)],
                        pdst_out.at[pl.ds(hbase + nf * fl + k * b_blk,
                                          b_blk)])

            cnt16 = jnp.where(lane == c, nf * fl + off, cnt16)

        cnt_v[pl.ds(0, 16)] = cnt16
        pltpu.sync_copy(cnt_v, ncnt_out.at[pl.ds(wid * 16, 16)])

    return part


def _make_drain(nch, ch_rows, sh_rows, fl, b_blk):
    zrows = sh_rows // NS
    orows = ch_rows // NS
    zfull = (zrows // 32) * 32
    blk_per_win = fl // b_blk

    @functools.partial(
        pl.kernel,
        out_type=jax.ShapeDtypeStruct((nch * ch_rows, D), jnp.float32),
        compiler_params=pltpu.CompilerParams(needs_layout_passes=False),
        mesh=_mesh,
        scratch_types=[
            pltpu.VMEM((32, D), jnp.float32),
            pltpu.VMEM((fl,), jnp.int32),
            pltpu.VMEM((fl,), jnp.int32),
            pltpu.VMEM((b_blk,), jnp.int32),
            pltpu.VMEM((b_blk,), jnp.int32),
            pltpu.VMEM((b_blk,), jnp.int32),
            pltpu.VMEM((b_blk,), jnp.int32),
            pltpu.VMEM((b_blk, D), jnp.float32),
            pltpu.VMEM((b_blk, D), jnp.float32),
            pltpu.VMEM((16,), jnp.int32),
            pltpu.VMEM_SHARED((sh_rows, D), jnp.float32),
            pltpu.SemaphoreType.DMA,
            pltpu.SemaphoreType.DMA,
            pltpu.SemaphoreType.DMA,
            pltpu.SemaphoreType.DMA,
        ],
    )
    def drain_kernel(x_src, psrc, pdst, ncnt, zeros, acc_out,
                     zbuf, swin, dwin, sf0, df0, sf1, df1,
                     rows0, rows1, cnt_v, sh_acc, sem0, sem1, ssem0, ssem1):
        core = lax.axis_index("c")
        sid = lax.axis_index("s")
        lane = lax.iota(jnp.int32, 16)
        pltpu.sync_copy(zeros, zbuf)

        fix = ((sf0, df0), (sf1, df1))
        rows = (rows0, rows1)
        sems = (sem0, sem1)
        ssems = (ssem0, ssem1)

        for ci in range(nch // NC):
            chunk = core + NC * ci

            @pl.loop(0, zfull, step=32)
            def _(r):
                pltpu.async_copy(zbuf.at[pl.ds(0, 32)],
                                 sh_acc.at[pl.ds(sid * zrows + r, 32)],
                                 ssem0)

            if zrows > zfull:
                pltpu.async_copy(zbuf.at[pl.ds(0, zrows - zfull)],
                                 sh_acc.at[pl.ds(sid * zrows + zfull,
                                                 zrows - zfull)],
                                 ssem0)

            @pl.loop(0, zfull, step=32)
            def _(r):
                pltpu.make_async_copy(zbuf.at[pl.ds(0, 32)],
                                      sh_acc.at[pl.ds(sid * zrows + r, 32)],
                                      ssem0).wait()

            if zrows > zfull:
                pltpu.make_async_copy(zbuf.at[pl.ds(0, zrows - zfull)],
                                      sh_acc.at[pl.ds(sid * zrows + zfull,
                                                      zrows - zfull)],
                                      ssem0).wait()

            plsc.subcore_barrier()

            for b in (2 * sid, 2 * sid + 1):
                pltpu.sync_copy(ncnt.at[pl.ds(b * 16, 16)], cnt_v)
                cnt = jnp.max(jnp.where(lane == chunk,
                                        cnt_v[pl.ds(0, 16)], 0))
                nb = (cnt + (b_blk - 1)) // b_blk
                seg = (b * nch + chunk) * P_CAP

                def fix_start(blk, koff, p):
                    @pl.when(blk >= 2)
                    def _():
                        pltpu.make_async_copy(rows[p],
                                              sh_acc.at[fix[p][1]],
                                              ssems[p]).wait()

                    @pl.loop(0, b_blk // 16)
                    def _(j):
                        pos = blk * b_blk + j * 16 + lane
                        ok = pos < cnt
                        s16 = swin[pl.ds(koff + j * 16, 16)]
                        d16 = dwin[pl.ds(koff + j * 16, 16)]
                        garb = ch_rows + lax.rem((b + j) * 16, G_GARB) + lane
                        fix[p][0][pl.ds(j * 16, 16)] = jnp.where(
                            ok, s16, j * 16 + lane)
                        fix[p][1][pl.ds(j * 16, 16)] = jnp.where(
                            ok, d16, garb)

                    pltpu.async_copy(x_src.at[fix[p][0]], rows[p], sems[p])

                def drain(p):
                    pltpu.make_async_copy(x_src.at[fix[p][0]], rows[p],
                                          sems[p]).wait()
                    pltpu.async_copy(rows[p], sh_acc.at[fix[p][1]],
                                     ssems[p], add=True)

                @pl.loop(0, P_CAP // fl)
                def _(w):
                    @pl.when(w * fl < cnt)
                    def _():
                        pltpu.sync_copy(psrc.at[pl.ds(seg + w * fl, fl)],
                                        swin)
                        pltpu.sync_copy(pdst.at[pl.ds(seg + w * fl, fl)],
                                        dwin)

                        @pl.loop(0, blk_per_win // 2)
                        def _(kp):
                            blk0 = w * blk_per_win + 2 * kp
                            blk1 = blk0 + 1

                            @pl.when(blk0 < nb)
                            def _():
                                fix_start(blk0, 2 * kp * b_blk, 0)

                            @pl.when(blk1 < nb)
                            def _():
                                fix_start(blk1, (2 * kp + 1) * b_blk, 1)

                            @pl.when(blk0 < nb)
                            def _():
                                drain(0)

                            @pl.when(blk1 < nb)
                            def _():
                                drain(1)

                @pl.when(nb >= 1)
                def _():
                    pltpu.make_async_copy(rows[0], sh_acc.at[fix[0][1]],
                                          ssems[0]).wait()

                @pl.when(nb >= 2)
                def _():
                    pltpu.make_async_copy(rows[1], sh_acc.at[fix[1][1]],
                                          ssems[1]).wait()

            plsc.subcore_barrier()
            pltpu.sync_copy(sh_acc.at[pl.ds(sid * orows, orows)],
                            acc_out.at[pl.ds(chunk * ch_rows + sid * orows,
                                             orows)])
            plsc.subcore_barrier()

    return drain_kernel


_sc_partition_gene = _make_partition(G_NCHUNK, G_CHUNK, FLUSH, B_GENE)
_sc_agg_gene = _make_drain(G_NCHUNK, G_CHUNK, G_SH_ROWS, FLUSH, B_GENE)


T_DIS = E // (NC * NS)
B_DIS2 = 80
NB_DIS = T_DIS // B_DIS2


@functools.partial(
    pl.kernel,
    out_type=jax.ShapeDtypeStruct((NC * D_PAD, D), jnp.float32),
    mesh=_mesh,
    scratch_types=[
        pltpu.VMEM((32, D), jnp.float32),
        pltpu.VMEM((T_DIS,), jnp.int32),
        pltpu.VMEM((T_DIS,), jnp.int32),
        pltpu.VMEM((B_DIS2,), jnp.int32),
        pltpu.VMEM((B_DIS2,), jnp.int32),
        pltpu.VMEM((B_DIS2, D), jnp.float32),
        pltpu.VMEM((B_DIS2, D), jnp.float32),
        pltpu.VMEM_SHARED((D_PAD, D), jnp.float32),
        pltpu.SemaphoreType.DMA,
        pltpu.SemaphoreType.DMA,
        pltpu.SemaphoreType.DMA,
        pltpu.SemaphoreType.DMA,
    ],
)
def _sc_agg_disease(x_src, src_idx, dst_idx, acc_out, zbuf, src_v, dst_v,
                    df0, df1, rows0, rows1, sh_acc, sem0, sem1, ssem0, ssem1):
    core = lax.axis_index("c")
    sid = lax.axis_index("s")
    _zero_vmem_2d(zbuf, 32)

    zrows = D_PAD // NS
    dfix = (df0, df1)
    rows = (rows0, rows1)
    sems = (sem0, sem1)
    ssems = (ssem0, ssem1)

    @pl.loop(0, zrows, step=32)
    def _(r):
        pltpu.async_copy(zbuf.at[pl.ds(0, 32)],
                         sh_acc.at[pl.ds(sid * zrows + r, 32)], ssem0)

    @pl.loop(0, zrows, step=32)
    def _(r):
        pltpu.make_async_copy(zbuf.at[pl.ds(0, 32)],
                              sh_acc.at[pl.ds(sid * zrows + r, 32)],
                              ssem0).wait()

    plsc.subcore_barrier()

    base = core * (E // NC) + sid * T_DIS
    pltpu.sync_copy(src_idx.at[pl.ds(base, T_DIS)], src_v)
    pltpu.sync_copy(dst_idx.at[pl.ds(base, T_DIS)], dst_v)

    def start(s, p, first):
        if not first:
            pltpu.make_async_copy(rows[p], sh_acc.at[dfix[p]],
                                  ssems[p]).wait()

        @pl.loop(0, B_DIS2 // 16)
        def _(j):
            dfix[p][pl.ds(j * 16, 16)] = dst_v[pl.ds(s * B_DIS2 + j * 16, 16)]

        pltpu.async_copy(x_src.at[src_v.at[pl.ds(s * B_DIS2, B_DIS2)]],
                         rows[p], sems[p])

    def drain(s, p):
        pltpu.make_async_copy(
            x_src.at[src_v.at[pl.ds(s * B_DIS2, B_DIS2)]],
            rows[p], sems[p]).wait()
        pltpu.async_copy(rows[p], sh_acc.at[dfix[p]], ssems[p], add=True)

    start(0, 0, True)
    start(1, 1, True)
    drain(0, 0)
    drain(1, 1)

    @pl.loop(1, NB_DIS // 2)
    def _(s2):
        start(2 * s2, 0, False)
        start(2 * s2 + 1, 1, False)
        drain(2 * s2, 0)
        drain(2 * s2 + 1, 1)

    start(NB_DIS - 1, 0, False)
    drain(NB_DIS - 1, 0)

    pltpu.make_async_copy(rows[0], sh_acc.at[dfix[0]], ssems[0]).wait()
    pltpu.make_async_copy(rows[1], sh_acc.at[dfix[1]], ssems[1]).wait()

    plsc.subcore_barrier()
    pltpu.sync_copy(sh_acc.at[pl.ds(sid * zrows, zrows)],
                    acc_out.at[pl.ds(core * D_PAD + sid * zrows, zrows)])


def _transform_body(two_acc, a0_ref, a1_ref, cnt_ref, x_ref, wl_ref, wr_ref,
                    b_ref, o_ref):
    if two_acc:
        agg = a0_ref[...] + a1_ref[...]
    else:
        agg = a0_ref[...]
        del a1_ref
    inv = 1.0 / jnp.maximum(cnt_ref[...], 1.0)
    agg = agg * inv
    dn = (((1,), (1,)), ((), ()))
    out = lax.dot_general(agg, wl_ref[...], dn,
                          precision=lax.Precision.HIGHEST,
                          preferred_element_type=jnp.float32)
    out += lax.dot_general(x_ref[...], wr_ref[...], dn,
                           precision=lax.Precision.HIGHEST,
                           preferred_element_type=jnp.float32)
    o_ref[...] = out + b_ref[...]


def _transform(n_rows, acc, partials, cnt, x_dst, w_l, w_r, b):
    if partials:
        bk = 1280
        grid = ((n_rows + bk - 1) // bk,)
        nblk = D_PAD // bk
        in_specs = [
            pl.BlockSpec((bk, D), lambda i: (i, 0)),
            pl.BlockSpec((bk, D), lambda i: (i + nblk, 0)),
        ]
        args = (acc, acc)
        body = functools.partial(_transform_body, True)
    else:
        bk = 2000
        grid = (n_rows // bk,)
        in_specs = [
            pl.BlockSpec((bk, D), lambda i: (i, 0)),
        ]
        args = (acc,)

        def body(a0, cnt_r, x_r, wl, wr, b_r, o_r):
            _transform_body(False, a0, None, cnt_r, x_r, wl, wr, b_r, o_r)
    in_specs += [
        pl.BlockSpec((bk, 1), lambda i: (i, 0)),
        pl.BlockSpec((bk, D), lambda i: (i, 0)),
        pl.BlockSpec((D, D), lambda i: (0, 0)),
        pl.BlockSpec((D, D), lambda i: (0, 0)),
        pl.BlockSpec((1, D), lambda i: (0, 0)),
    ]
    return pl.pallas_call(
        body,
        grid=grid,
        in_specs=in_specs,
        out_specs=pl.BlockSpec((bk, D), lambda i: (i, 0)),
        out_shape=jax.ShapeDtypeStruct((n_rows, D), jnp.float32),
    )(*args, cnt, x_dst, w_l, w_r, b)


def kernel(x_disease, x_gene, src_dg, dst_dg, src_gd, dst_gd,
           W_l_dg1, W_r_dg1, W_l_gd1, W_r_gd1,
           W_l_dg2, W_r_dg2, W_l_gd2, W_r_gd2,
           b_dg1, b_gd1, b_dg2, b_gd2):
    cnt_g, cnt_d = _sc_counts(dst_dg, dst_gd)
    cnt_g = cnt_g.reshape(G_CNT_PAD, 1)
    cnt_d = cnt_d.reshape(D_PAD, 1)
    psrc_g, pdst_g, ncnt_g = _sc_partition_gene(src_dg, dst_dg)

    zeros = jnp.zeros((32, D), jnp.float32)
    acc_g1 = _sc_agg_gene(x_disease, psrc_g, pdst_g, ncnt_g, zeros)
    acc_d1 = _sc_agg_disease(x_gene, src_gd, dst_gd)
    g1 = _transform(N_G, acc_g1, False, cnt_g, x_gene,
                    W_l_dg1, W_r_dg1, b_dg1.reshape(1, D))
    d1 = _transform(N_D, acc_d1, True, cnt_d, x_disease,
                    W_l_gd1, W_r_gd1, b_gd1.reshape(1, D))

    acc_g2 = _sc_agg_gene(d1, psrc_g, pdst_g, ncnt_g, zeros)
    acc_d2 = _sc_agg_disease(g1, src_gd, dst_gd)
    g2 = _transform(N_G, acc_g2, False, cnt_g, g1,
                    W_l_dg2, W_r_dg2, b_dg2.reshape(1, D))
    d2 = _transform(N_D, acc_d2, True, cnt_d, d1,
                    W_l_gd2, W_r_gd2, b_gd2.reshape(1, D))
    return (d2, g2)


# speedup vs baseline: 5.7730x; 1.0026x over previous
import functools

import jax
import jax.numpy as jnp
from jax import lax
from jax.experimental import pallas as pl
from jax.experimental.pallas import tpu as pltpu
from jax.experimental.pallas import tpu_sc as plsc

N_D = 10000
N_G = 50000
E = 320000
D = 128

NC = 2
NS = 16

G_CHUNK = 8192
G_NCHUNK = 8
G_PAD = G_CHUNK * G_NCHUNK
G_GARB = 256
G_SH_ROWS = G_CHUNK + G_GARB
G_CNT_PAD = 51200

D_PAD = 10240

W_CNT = 2000
W_PART = 2000
T_EDGE = E // 32
P_CAP = 10240
FLUSH = 1280
B_GENE = 160

_mesh = plsc.VectorSubcoreMesh(core_axis_name="c", subcore_axis_name="s")

def _zero_vmem_2d(buf, rows):
    z = jnp.zeros((16,), jnp.float32)

    @pl.loop(0, rows)
    def _(r):
        for l in range(8):
            buf[r, pl.ds(l * 16, 16)] = z


def _fill_vmem_1d(buf, n, value):
    v = jnp.full((16,), value, jnp.float32)

    @pl.loop(0, n, step=16)
    def _(i):
        buf[pl.ds(i, 16)] = v


@functools.partial(
    pl.kernel,
    out_type=(
        jax.ShapeDtypeStruct((G_CNT_PAD,), jnp.float32),
        jax.ShapeDtypeStruct((D_PAD,), jnp.float32),
    ),
    mesh=_mesh,
    scratch_types=[
        pltpu.VMEM((3200,), jnp.float32),
        pltpu.VMEM((W_CNT,), jnp.float32),
        pltpu.VMEM((W_CNT,), jnp.int32),
        pltpu.VMEM_SHARED((G_CNT_PAD,), jnp.float32),
        pltpu.VMEM_SHARED((D_PAD,), jnp.float32),
    ],
)
def _sc_counts(dst_dg, dst_gd, cnt_g_out, cnt_d_out, zbuf, ones_v, idx_v,
               sh_g, sh_d):
    core = lax.axis_index("c")
    sid = lax.axis_index("s")
    _fill_vmem_1d(zbuf, 3200, 0.0)
    _fill_vmem_1d(ones_v, W_CNT, 1.0)

    @pl.when(core == 0)
    def _():
        pltpu.sync_copy(zbuf.at[pl.ds(0, G_CNT_PAD // NS)],
                        sh_g.at[pl.ds(sid * (G_CNT_PAD // NS), G_CNT_PAD // NS)])

    @pl.when(core == 1)
    def _():
        pltpu.sync_copy(zbuf.at[pl.ds(0, D_PAD // NS)],
                        sh_d.at[pl.ds(sid * (D_PAD // NS), D_PAD // NS)])

    plsc.subcore_barrier()

    per_tile = E // NS

    @pl.loop(0, per_tile // W_CNT)
    def _(w):
        base = sid * per_tile + w * W_CNT

        @pl.when(core == 0)
        def _():
            pltpu.sync_copy(dst_dg.at[pl.ds(base, W_CNT)], idx_v)
            pltpu.sync_copy(ones_v, sh_g.at[idx_v], add=True)

        @pl.when(core == 1)
        def _():
            pltpu.sync_copy(dst_gd.at[pl.ds(base, W_CNT)], idx_v)
            pltpu.sync_copy(ones_v, sh_d.at[idx_v], add=True)

    plsc.subcore_barrier()

    @pl.when(core == 0)
    def _():
        pltpu.sync_copy(sh_g.at[pl.ds(sid * (G_CNT_PAD // NS), G_CNT_PAD // NS)],
                        cnt_g_out.at[pl.ds(sid * (G_CNT_PAD // NS), G_CNT_PAD // NS)])

    @pl.when(core == 1)
    def _():
        pltpu.sync_copy(sh_d.at[pl.ds(sid * (D_PAD // NS), D_PAD // NS)],
                        cnt_d_out.at[pl.ds(sid * (D_PAD // NS), D_PAD // NS)])


def _make_partition(nch, ch_rows, fl, b_blk):
    @functools.partial(
        pl.kernel,
        out_type=(
            jax.ShapeDtypeStruct((32 * nch * P_CAP,), jnp.int32),
            jax.ShapeDtypeStruct((32 * nch * P_CAP,), jnp.int32),
            jax.ShapeDtypeStruct((32 * 16,), jnp.int32),
        ),
        compiler_params=pltpu.CompilerParams(needs_layout_passes=False),
        mesh=_mesh,
        scratch_types=[
            pltpu.VMEM((W_PART,), jnp.int32),
            pltpu.VMEM((W_PART,), jnp.int32),
            pltpu.VMEM((nch * (fl + 16),), jnp.int32),
            pltpu.VMEM((nch * (fl + 16),), jnp.int32),
            pltpu.VMEM((16,), jnp.int32),
        ],
    )
    def part(src_idx, dst_idx, psrc_out, pdst_out, ncnt_out,
             src_v, dst_v, bsrc, bdst, cnt_v):
        core = lax.axis_index("c")
        sid = lax.axis_index("s")
        wid = sid * NC + core
        lane = lax.iota(jnp.int32, 16)
        ebase = wid * T_EDGE

        def _mkflush(c):
            vbase = c * (fl + 16)

            def flush(off, nf):
                hbase = (wid * nch + c) * P_CAP
                pltpu.sync_copy(bsrc.at[pl.ds(vbase, fl)],
                                psrc_out.at[pl.ds(hbase + nf * fl, fl)])
                pltpu.sync_copy(bdst.at[pl.ds(vbase, fl)],
                                pdst_out.at[pl.ds(hbase + nf * fl, fl)])
                bsrc[pl.ds(vbase, 16)] = bsrc[pl.ds(vbase + fl, 16)]
                bdst[pl.ds(vbase, 16)] = bdst[pl.ds(vbase + fl, 16)]
                return off - fl, nf + 1

            def keep(off, nf):
                return off, nf

            return flush, keep

        def vreg(j, carry):
            offs, nfs = carry
            d16 = dst_v[pl.ds(j * 16, 16)]
            s16 = src_v[pl.ds(j * 16, 16)]
            new_offs, new_nfs = [], []
            for c in range(nch):
                lo = c * ch_rows
                m = (d16 >= lo) & (d16 < lo + ch_rows)
                off, nf = offs[c], nfs[c]
                vbase = c * (fl + 16)
                plsc.store_compressed(bdst.at[pl.ds(vbase + off, 16)],
                                      d16 - lo, mask=m)
                plsc.store_compressed(bsrc.at[pl.ds(vbase + off, 16)],
                                      s16, mask=m)
                off = off + jnp.max(plsc.all_reduce_population_count(m))
                flush, keep = _mkflush(c)
                off, nf = lax.cond(off >= fl, flush, keep, off, nf)
                new_offs.append(off)
                new_nfs.append(nf)
            return tuple(new_offs), tuple(new_nfs)

        def window(w, carry):
            pltpu.sync_copy(src_idx.at[pl.ds(ebase + w * W_PART, W_PART)],
                            src_v)
            pltpu.sync_copy(dst_idx.at[pl.ds(ebase + w * W_PART, W_PART)],
                            dst_v)
            return lax.fori_loop(0, W_PART // 16, vreg, carry)

        zero = jnp.int32(0)
        offs, nfs = lax.fori_loop(0, T_EDGE // W_PART, window,
                                  ((zero,) * nch, (zero,) * nch))

        cnt16 = jnp.zeros((16,), jnp.int32)
        for c in range(nch):
            off, nf = offs[c], nfs[c]
            vbase = c * (fl + 16)
            hbase = (wid * nch + c) * P_CAP
            for k in range(fl // b_blk):
                @pl.when(k * b_blk < off)
                def _():
                    pltpu.sync_copy(
                        bsrc.at[pl.ds(vbase + k * b_blk, b_blk)],
                        psrc_out.at[pl.ds(hbase + nf * fl + k * b_blk,
                                          b_blk)])
                    pltpu.sync_copy(
                        bdst.at[pl.ds(vbase + k * b_blk, b_blk)],
                        pdst_out.at[pl.ds(hbase + nf * fl + k * b_blk,
                                          b_blk)])

            cnt16 = jnp.where(lane == c, nf * fl + off, cnt16)

        cnt_v[pl.ds(0, 16)] = cnt16
        pltpu.sync_copy(cnt_v, ncnt_out.at[pl.ds(wid * 16, 16)])

    return part


def _make_drain(nch, ch_rows, sh_rows, fl, b_blk):
    zrows = sh_rows // NS
    orows = ch_rows // NS
    zfull = (zrows // 32) * 32
    blk_per_win = fl // b_blk

    @functools.partial(
        pl.kernel,
        out_type=jax.ShapeDtypeStruct((nch * ch_rows, D), jnp.float32),
        compiler_params=pltpu.CompilerParams(needs_layout_passes=False),
        mesh=_mesh,
        scratch_types=[
            pltpu.VMEM((32, D), jnp.float32),
            pltpu.VMEM((fl,), jnp.int32),
            pltpu.VMEM((fl,), jnp.int32),
            pltpu.VMEM((b_blk,), jnp.int32),
            pltpu.VMEM((b_blk,), jnp.int32),
            pltpu.VMEM((b_blk,), jnp.int32),
            pltpu.VMEM((b_blk,), jnp.int32),
            pltpu.VMEM((b_blk, D), jnp.float32),
            pltpu.VMEM((b_blk, D), jnp.float32),
            pltpu.VMEM((16,), jnp.int32),
            pltpu.VMEM_SHARED((sh_rows, D), jnp.float32),
            pltpu.SemaphoreType.DMA,
            pltpu.SemaphoreType.DMA,
            pltpu.SemaphoreType.DMA,
            pltpu.SemaphoreType.DMA,
        ],
    )
    def drain_kernel(x_src, psrc, pdst, ncnt, zeros, acc_out,
                     zbuf, swin, dwin, sf0, df0, sf1, df1,
                     rows0, rows1, cnt_v, sh_acc, sem0, sem1, ssem0, ssem1):
        core = lax.axis_index("c")
        sid = lax.axis_index("s")
        lane = lax.iota(jnp.int32, 16)
        pltpu.sync_copy(zeros, zbuf)

        fix = ((sf0, df0), (sf1, df1))
        rows = (rows0, rows1)
        sems = (sem0, sem1)
        ssems = (ssem0, ssem1)

        for ci in range(nch // NC):
            chunk = core + NC * ci

            @pl.loop(0, zfull, step=32)
            def _(r):
                pltpu.async_copy(zbuf.at[pl.ds(0, 32)],
                                 sh_acc.at[pl.ds(sid * zrows + r, 32)],
                                 ssem0)

            if zrows > zfull:
                pltpu.async_copy(zbuf.at[pl.ds(0, zrows - zfull)],
                                 sh_acc.at[pl.ds(sid * zrows + zfull,
                                                 zrows - zfull)],
                                 ssem0)

            @pl.loop(0, zfull, step=32)
            def _(r):
                pltpu.make_async_copy(zbuf.at[pl.ds(0, 32)],
                                      sh_acc.at[pl.ds(sid * zrows + r, 32)],
                                      ssem0).wait()

            if zrows > zfull:
                pltpu.make_async_copy(zbuf.at[pl.ds(0, zrows - zfull)],
                                      sh_acc.at[pl.ds(sid * zrows + zfull,
                                                      zrows - zfull)],
                                      ssem0).wait()

            plsc.subcore_barrier()

            for b in (2 * sid, 2 * sid + 1):
                pltpu.sync_copy(ncnt.at[pl.ds(b * 16, 16)], cnt_v)
                cnt = jnp.max(jnp.where(lane == chunk,
                                        cnt_v[pl.ds(0, 16)], 0))
                nb = (cnt + (b_blk - 1)) // b_blk
                seg = (b * nch + chunk) * P_CAP

                def fix_start(blk, koff, p):
                    @pl.when(blk >= 2)
                    def _():
                        pltpu.make_async_copy(rows[p],
                                              sh_acc.at[fix[p][1]],
                                              ssems[p]).wait()

                    @pl.loop(0, b_blk // 16)
                    def _(j):
                        pos = blk * b_blk + j * 16 + lane
                        ok = pos < cnt
                        s16 = swin[pl.ds(koff + j * 16, 16)]
                        d16 = dwin[pl.ds(koff + j * 16, 16)]
                        garb = ch_rows + lax.rem((b + j) * 16, G_GARB) + lane
                        fix[p][0][pl.ds(j * 16, 16)] = jnp.where(
                            ok, s16, j * 16 + lane)
                        fix[p][1][pl.ds(j * 16, 16)] = jnp.where(
                            ok, d16, garb)

                    pltpu.async_copy(x_src.at[fix[p][0]], rows[p], sems[p])

                def drain(p):
                    pltpu.make_async_copy(x_src.at[fix[p][0]], rows[p],
                                          sems[p]).wait()
                    pltpu.async_copy(rows[p], sh_acc.at[fix[p][1]],
                                     ssems[p], add=True)

                @pl.loop(0, P_CAP // fl)
                def _(w):
                    @pl.when(w * fl < cnt)
                    def _():
                        pltpu.sync_copy(psrc.at[pl.ds(seg + w * fl, fl)],
                                        swin)
                        pltpu.sync_copy(pdst.at[pl.ds(seg + w * fl, fl)],
                                        dwin)

                        @pl.loop(0, blk_per_win // 2)
                        def _(kp):
                            blk0 = w * blk_per_win + 2 * kp
                            blk1 = blk0 + 1

                            @pl.when(blk0 < nb)
                            def _():
                                fix_start(blk0, 2 * kp * b_blk, 0)

                            @pl.when(blk1 < nb)
                            def _():
                                fix_start(blk1, (2 * kp + 1) * b_blk, 1)

                            @pl.when(blk0 < nb)
                            def _():
                                drain(0)

                            @pl.when(blk1 < nb)
                            def _():
                                drain(1)

                @pl.when(nb >= 1)
                def _():
                    pltpu.make_async_copy(rows[0], sh_acc.at[fix[0][1]],
                                          ssems[0]).wait()

                @pl.when(nb >= 2)
                def _():
                    pltpu.make_async_copy(rows[1], sh_acc.at[fix[1][1]],
                                          ssems[1]).wait()

            plsc.subcore_barrier()
            pltpu.sync_copy(sh_acc.at[pl.ds(sid * orows, orows)],
                            acc_out.at[pl.ds(chunk * ch_rows + sid * orows,
                                             orows)])
            plsc.subcore_barrier()

    return drain_kernel


_sc_partition_gene = _make_partition(G_NCHUNK, G_CHUNK, FLUSH, B_GENE)
_sc_agg_gene = _make_drain(G_NCHUNK, G_CHUNK, G_SH_ROWS, FLUSH, B_GENE)


T_DIS = E // (NC * NS)
B_DIS2 = 80
NB_DIS = T_DIS // B_DIS2


@functools.partial(
    pl.kernel,
    out_type=jax.ShapeDtypeStruct((NC * D_PAD, D), jnp.float32),
    mesh=_mesh,
    scratch_types=[
        pltpu.VMEM((32, D), jnp.float32),
        pltpu.VMEM((T_DIS,), jnp.int32),
        pltpu.VMEM((T_DIS,), jnp.int32),
        pltpu.VMEM((B_DIS2,), jnp.int32),
        pltpu.VMEM((B_DIS2,), jnp.int32),
        pltpu.VMEM((B_DIS2, D), jnp.float32),
        pltpu.VMEM((B_DIS2, D), jnp.float32),
        pltpu.VMEM_SHARED((D_PAD, D), jnp.float32),
        pltpu.SemaphoreType.DMA,
        pltpu.SemaphoreType.DMA,
        pltpu.SemaphoreType.DMA,
        pltpu.SemaphoreType.DMA,
    ],
)
def _sc_agg_disease(x_src, src_idx, dst_idx, acc_out, zbuf, src_v, dst_v,
                    df0, df1, rows0, rows1, sh_acc, sem0, sem1, ssem0, ssem1):
    core = lax.axis_index("c")
    sid = lax.axis_index("s")
    _zero_vmem_2d(zbuf, 32)

    zrows = D_PAD // NS
    dfix = (df0, df1)
    rows = (rows0, rows1)
    sems = (sem0, sem1)
    ssems = (ssem0, ssem1)

    @pl.loop(0, zrows, step=32)
    def _(r):
        pltpu.async_copy(zbuf.at[pl.ds(0, 32)],
                         sh_acc.at[pl.ds(sid * zrows + r, 32)], ssem0)

    @pl.loop(0, zrows, step=32)
    def _(r):
        pltpu.make_async_copy(zbuf.at[pl.ds(0, 32)],
                              sh_acc.at[pl.ds(sid * zrows + r, 32)],
                              ssem0).wait()

    plsc.subcore_barrier()

    base = core * (E // NC) + sid * T_DIS
    pltpu.sync_copy(src_idx.at[pl.ds(base, T_DIS)], src_v)
    pltpu.sync_copy(dst_idx.at[pl.ds(base, T_DIS)], dst_v)

    def start(s, p, first):
        if not first:
            pltpu.make_async_copy(rows[p], sh_acc.at[dfix[p]],
                                  ssems[p]).wait()

        @pl.loop(0, B_DIS2 // 16)
        def _(j):
            dfix[p][pl.ds(j * 16, 16)] = dst_v[pl.ds(s * B_DIS2 + j * 16, 16)]

        pltpu.async_copy(x_src.at[src_v.at[pl.ds(s * B_DIS2, B_DIS2)]],
                         rows[p], sems[p])

    def drain(s, p):
        pltpu.make_async_copy(
            x_src.at[src_v.at[pl.ds(s * B_DIS2, B_DIS2)]],
            rows[p], sems[p]).wait()
        pltpu.async_copy(rows[p], sh_acc.at[dfix[p]], ssems[p], add=True)

    start(0, 0, True)
    start(1, 1, True)
    drain(0, 0)
    drain(1, 1)

    @pl.loop(1, NB_DIS // 2)
    def _(s2):
        start(2 * s2, 0, False)
        start(2 * s2 + 1, 1, False)
        drain(2 * s2, 0)
        drain(2 * s2 + 1, 1)

    start(NB_DIS - 1, 0, False)
    drain(NB_DIS - 1, 0)

    pltpu.make_async_copy(rows[0], sh_acc.at[dfix[0]], ssems[0]).wait()
    pltpu.make_async_copy(rows[1], sh_acc.at[dfix[1]], ssems[1]).wait()

    plsc.subcore_barrier()
    pltpu.sync_copy(sh_acc.at[pl.ds(sid * zrows, zrows)],
                    acc_out.at[pl.ds(core * D_PAD + sid * zrows, zrows)])


def _transform_body(two_acc, a0_ref, a1_ref, cnt_ref, x_ref, wl_ref, wr_ref,
                    b_ref, o_ref):
    if two_acc:
        agg = a0_ref[...] + a1_ref[...]
    else:
        agg = a0_ref[...]
        del a1_ref
    inv = 1.0 / jnp.maximum(cnt_ref[...], 1.0)
    agg = agg * inv
    dn = (((1,), (1,)), ((), ()))
    out = lax.dot_general(agg, wl_ref[...], dn,
                          precision=lax.Precision.HIGHEST,
                          preferred_element_type=jnp.float32)
    out += lax.dot_general(x_ref[...], wr_ref[...], dn,
                           precision=lax.Precision.HIGHEST,
                           preferred_element_type=jnp.float32)
    o_ref[...] = out + b_ref[...]


def _transform(n_rows, acc, partials, cnt, x_dst, w_l, w_r, b):
    if partials:
        bk = 1280
        grid = ((n_rows + bk - 1) // bk,)
        nblk = D_PAD // bk
        in_specs = [
            pl.BlockSpec((bk, D), lambda i: (i, 0)),
            pl.BlockSpec((bk, D), lambda i: (i + nblk, 0)),
        ]
        args = (acc, acc)
        body = functools.partial(_transform_body, True)
    else:
        bk = 2000
        grid = (n_rows // bk,)
        in_specs = [
            pl.BlockSpec((bk, D), lambda i: (i, 0)),
        ]
        args = (acc,)

        def body(a0, cnt_r, x_r, wl, wr, b_r, o_r):
            _transform_body(False, a0, None, cnt_r, x_r, wl, wr, b_r, o_r)
    in_specs += [
        pl.BlockSpec((bk, 1), lambda i: (i, 0)),
        pl.BlockSpec((bk, D), lambda i: (i, 0)),
        pl.BlockSpec((D, D), lambda i: (0, 0)),
        pl.BlockSpec((D, D), lambda i: (0, 0)),
        pl.BlockSpec((1, D), lambda i: (0, 0)),
    ]
    return pl.pallas_call(
        body,
        grid=grid,
        in_specs=in_specs,
        out_specs=pl.BlockSpec((bk, D), lambda i: (i, 0)),
        out_shape=jax.ShapeDtypeStruct((n_rows, D), jnp.float32),
    )(*args, cnt, x_dst, w_l, w_r, b)


def kernel(x_disease, x_gene, src_dg, dst_dg, src_gd, dst_gd,
           W_l_dg1, W_r_dg1, W_l_gd1, W_r_gd1,
           W_l_dg2, W_r_dg2, W_l_gd2, W_r_gd2,
           b_dg1, b_gd1, b_dg2, b_gd2):
    cnt_g, cnt_d = _sc_counts(dst_dg, dst_gd)
    cnt_g = cnt_g.reshape(G_CNT_PAD, 1)
    cnt_d = cnt_d.reshape(D_PAD, 1)
    psrc_g, pdst_g, ncnt_g = _sc_partition_gene(src_dg, dst_dg)

    zeros = jnp.zeros((32, D), jnp.float32)
    acc_g1 = _sc_agg_gene(x_disease, psrc_g, pdst_g, ncnt_g, zeros)
    acc_d1 = _sc_agg_disease(x_gene, src_gd, dst_gd)
    g1 = _transform(N_G, acc_g1, False, cnt_g, x_gene,
                    W_l_dg1, W_r_dg1, b_dg1.reshape(1, D))
    d1 = _transform(N_D, acc_d1, True, cnt_d, x_disease,
                    W_l_gd1, W_r_gd1, b_gd1.reshape(1, D))

    acc_g2 = _sc_agg_gene(d1, psrc_g, pdst_g, ncnt_g, zeros)
    acc_d2 = _sc_agg_disease(g1, src_gd, dst_gd)
    g2 = _transform(N_G, acc_g2, False, cnt_g, g1,
                    W_l_dg2, W_r_dg2, b_dg2.reshape(1, D))
    d2 = _transform(N_D, acc_d2, True, cnt_d, d1,
                    W_l_gd2, W_r_gd2, b_gd2.reshape(1, D))
    return (d2, g2)
